# baseline (speedup 1.0000x reference)
import functools

import jax
import jax.numpy as jnp
from jax import lax
from jax.experimental import pallas as pl
from jax.experimental.pallas import tpu as pltpu
from jax.experimental.pallas import tpu_sc as plsc

NC = 2
NS = 16
NW = NC * NS
LANES = 16
K = 128


def _mesh():
    return plsc.VectorSubcoreMesh(
        core_axis_name="c", subcore_axis_name="s", num_cores=NC, num_subcores=NS
    )


def _deg_body(n_acc, ep, dst3, out, didx, ones_v, zvec, deg_acc):
    c = lax.axis_index("c")
    s = lax.axis_index("s")
    wid = c * NS + s
    rp = n_acc // NS

    pltpu.sync_copy(dst3.at[wid], didx)

    def fill(i, _):
        zvec[pl.ds(i * LANES, LANES)] = jnp.zeros((LANES,), jnp.float32)
        ones_v[pl.ds((i % (K // LANES)) * LANES, LANES)] = jnp.ones(
            (LANES,), jnp.float32
        )
        return 0

    lax.fori_loop(0, rp // LANES, fill, 0)
    pltpu.sync_copy(zvec, deg_acc.at[pl.ds(s * rp, rp)])
    plsc.subcore_barrier()

    def chunk(k, _):
        pltpu.sync_copy(ones_v, deg_acc.at[didx.at[k]], add=True)
        return 0

    lax.fori_loop(0, ep // K, chunk, 0)
    plsc.subcore_barrier()
    pltpu.sync_copy(deg_acc.at[pl.ds(s * rp, rp)], out.at[c, pl.ds(s * rp, rp)])


def _agg_body(
    n_acc, ep, d, g, srcp, dstp, out,
    sidx, didx_a, didx_b, rows_a, rows_b, gsem_a, gsem_b, acc,
):
    c = lax.axis_index("c")
    s = lax.axis_index("s")
    wid = c * NS + s
    rp = n_acc // NS
    nch = ep // K

    pltpu.sync_copy(srcp.at[pl.ds(pl.multiple_of(wid * ep, 8), ep)], sidx)

    def fill(i, _):
        rows_a[i // (d // LANES), pl.ds((i % (d // LANES)) * LANES, LANES)] = (
            jnp.zeros((LANES,), jnp.float32)
        )
        return 0

    lax.fori_loop(0, K * d // LANES, fill, 0)
    for z in range(rp // K):
        pltpu.sync_copy(rows_a, acc.at[pl.ds(s * rp + z * K, K)])
    plsc.subcore_barrier()

    def gather(chunk, buf, sem):
        off = pl.multiple_of(chunk * K, 8)
        pltpu.async_copy(g.at[sidx.at[pl.ds(off, K)]], buf, sem)

    def gwait(buf, sem):
        pltpu.make_async_copy(g.at[sidx.at[pl.ds(0, K)]], buf, sem).wait()

    def dload(chunk, didx):
        off = pl.multiple_of(wid * ep + chunk * K, 8)
        pltpu.sync_copy(dstp.at[pl.ds(off, K)], didx)

    def scatter(buf, didx):
        pltpu.sync_copy(buf, acc.at[didx], add=True)

    dload(0, didx_a)
    gather(0, rows_a, gsem_a)

    def body(p, _):
        k0 = 2 * p
        dload(k0 + 1, didx_b)
        gather(k0 + 1, rows_b, gsem_b)
        gwait(rows_a, gsem_a)
        scatter(rows_a, didx_a)

        @pl.when(p < nch // 2 - 1)
        def _():
            dload(k0 + 2, didx_a)
            gather(k0 + 2, rows_a, gsem_a)

        gwait(rows_b, gsem_b)
        scatter(rows_b, didx_b)
        return 0

    lax.fori_loop(0, nch // 2, body, 0)
    plsc.subcore_barrier()
    pltpu.sync_copy(acc.at[pl.ds(s * rp, rp)], out.at[c, pl.ds(s * rp, rp)])


def _tca_body(x_ref, w_ref, degp_ref, g_ref):
    deg = degp_ref[0, :] + degp_ref[1, :] + 1.0
    dinv = lax.rsqrt(deg)[:, None]
    h = jnp.dot(x_ref[...], w_ref[...], preferred_element_type=jnp.float32)
    g_ref[...] = h * dinv


def _tcb_body(p_ref, g_ref, degp_ref, w_ref, b_ref, out_ref):
    deg = degp_ref[0, :] + degp_ref[1, :] + 1.0
    dinv = lax.rsqrt(deg)[:, None]
    t = dinv * (p_ref[0] + p_ref[1] + g_ref[...]) + b_ref[...]
    o1 = jnp.where(t >= 0, t, 0.01 * t)
    h2 = jnp.dot(o1, w_ref[...], preferred_element_type=jnp.float32)
    out_ref[...] = h2 * dinv


def _tcc_body(p_ref, g_ref, degp_ref, b_ref, out_ref):
    deg = degp_ref[0, :] + degp_ref[1, :] + 1.0
    dinv = lax.rsqrt(deg)[:, None]
    t = dinv * (p_ref[0] + p_ref[1] + g_ref[...]) + b_ref[...]
    out_ref[...] = jnp.where(t >= 0, t, 0.01 * t)


def kernel(x, edge_index, batch, W1, b1, W2, b2):
    n, d = x.shape
    e = edge_index.shape[1]

    ch = 2 * K
    ep = -(-e // (NW * ch)) * ch
    e_pad = ep * NW
    nch = ep // K
    n_acc = -(-n // (NS * K)) * NS * K
    src = edge_index[0].astype(jnp.int32)
    dst = edge_index[1].astype(jnp.int32)
    pad = e_pad - e
    srcp = jnp.concatenate([src, jnp.zeros((pad,), jnp.int32)])
    dstp = jnp.concatenate([dst, jnp.full((pad,), n, jnp.int32)])
    dst3 = dstp.reshape(NW, nch, K)
    xp = jnp.concatenate([x, jnp.zeros((n_acc - n, d), x.dtype)])

    deg_kernel = pl.kernel(
        functools.partial(_deg_body, n_acc, ep),
        out_type=jax.ShapeDtypeStruct((NC, n_acc), jnp.float32),
        mesh=_mesh(),
        scratch_types={
            "didx": pltpu.VMEM((nch, K), jnp.int32),
            "ones_v": pltpu.VMEM((K,), jnp.float32),
            "zvec": pltpu.VMEM((n_acc // NS,), jnp.float32),
            "deg_acc": pltpu.MemorySpace.VMEM_SHARED((n_acc,), jnp.float32),
        },
        name="gcn_sc_degree",
    )

    agg_kernel = pl.kernel(
        functools.partial(_agg_body, n_acc, ep, d),
        out_type=jax.ShapeDtypeStruct((NC, n_acc, d), jnp.float32),
        mesh=_mesh(),
        scratch_types={
            "sidx": pltpu.VMEM((ep,), jnp.int32),
            "didx_a": pltpu.VMEM((K,), jnp.int32),
            "didx_b": pltpu.VMEM((K,), jnp.int32),
            "rows_a": pltpu.VMEM((K, d), jnp.float32),
            "rows_b": pltpu.VMEM((K, d), jnp.float32),
            "gsem_a": pltpu.SemaphoreType.DMA,
            "gsem_b": pltpu.SemaphoreType.DMA,
            "acc": pltpu.MemorySpace.VMEM_SHARED((n_acc, d), jnp.float32),
        },
        name="gcn_sc_scatter",
    )

    br = 2048
    grid = (n_acc // br,)
    tca = pl.pallas_call(
        _tca_body,
        grid=grid,
        in_specs=[
            pl.BlockSpec((br, d), lambda i: (i, 0)),
            pl.BlockSpec((d, d), lambda i: (0, 0)),
            pl.BlockSpec((NC, br), lambda i: (0, i)),
        ],
        out_specs=pl.BlockSpec((br, d), lambda i: (i, 0)),
        out_shape=jax.ShapeDtypeStruct((n_acc, d), jnp.float32),
        name="gcn_tc_g1",
    )
    tcb = pl.pallas_call(
        _tcb_body,
        grid=grid,
        in_specs=[
            pl.BlockSpec((NC, br, d), lambda i: (0, i, 0)),
            pl.BlockSpec((br, d), lambda i: (i, 0)),
            pl.BlockSpec((NC, br), lambda i: (0, i)),
            pl.BlockSpec((d, d), lambda i: (0, 0)),
            pl.BlockSpec((1, d), lambda i: (0, 0)),
        ],
        out_specs=pl.BlockSpec((br, d), lambda i: (i, 0)),
        out_shape=jax.ShapeDtypeStruct((n_acc, d), jnp.float32),
        name="gcn_tc_layer1",
    )
    tcc = pl.pallas_call(
        _tcc_body,
        grid=grid,
        in_specs=[
            pl.BlockSpec((NC, br, d), lambda i: (0, i, 0)),
            pl.BlockSpec((br, d), lambda i: (i, 0)),
            pl.BlockSpec((NC, br), lambda i: (0, i)),
            pl.BlockSpec((1, d), lambda i: (0, 0)),
        ],
        out_specs=pl.BlockSpec((br, d), lambda i: (i, 0)),
        out_shape=jax.ShapeDtypeStruct((n_acc, d), jnp.float32),
        name="gcn_tc_layer2",
    )

    degp = deg_kernel(dst3)
    g1 = tca(xp, W1, degp)
    p1 = agg_kernel(g1, srcp, dstp)
    g2 = tcb(p1, g1, degp, W2, b1.reshape(1, d))
    p2 = agg_kernel(g2, srcp, dstp)
    out = tcc(p2, g2, degp, b2.reshape(1, d))
    return out[:n]


# speedup vs baseline: 1.2128x; 1.2128x over previous
import functools

import jax
import jax.numpy as jnp
from jax import lax
from jax.experimental import pallas as pl
from jax.experimental.pallas import tpu as pltpu
from jax.experimental.pallas import tpu_sc as plsc

NC = 2
NS = 16
NW = NC * NS
LANES = 16
K = 32
KD = 128


def _mesh():
    return plsc.VectorSubcoreMesh(
        core_axis_name="c", subcore_axis_name="s", num_cores=NC, num_subcores=NS
    )


def _deg_body(n_acc, ep, dst3, out, didx, ones_v, zvec, deg_acc):
    c = lax.axis_index("c")
    s = lax.axis_index("s")
    wid = c * NS + s
    rp = n_acc // NS

    pltpu.sync_copy(dst3.at[wid], didx)

    def fill(i, _):
        zvec[pl.ds(i * LANES, LANES)] = jnp.zeros((LANES,), jnp.float32)
        ones_v[pl.ds((i % (KD // LANES)) * LANES, LANES)] = jnp.ones(
            (LANES,), jnp.float32
        )
        return 0

    lax.fori_loop(0, rp // LANES, fill, 0)
    pltpu.sync_copy(zvec, deg_acc.at[pl.ds(s * rp, rp)])
    plsc.subcore_barrier()

    def chunk(k, _):
        pltpu.sync_copy(ones_v, deg_acc.at[didx.at[k]], add=True)
        return 0

    lax.fori_loop(0, ep // KD, chunk, 0)
    plsc.subcore_barrier()
    pltpu.sync_copy(deg_acc.at[pl.ds(s * rp, rp)], out.at[c, pl.ds(s * rp, rp)])


def _agg_body(
    n, hn, na, ept, d, g, eidx, out,
    ebuf_a, ebuf_b, didx_a, didx_b, rows_a, rows_b,
    esem_a, esem_b, gsem_a, gsem_b, tbl, acc,
):
    c = lax.axis_index("c")
    s = lax.axis_index("s")
    nch = ept // K
    trp = (n // NS) // 8 * 8
    arp = hn // NS
    base = c * hn

    toff = pl.multiple_of(s * trp, 8)
    pltpu.sync_copy(g.at[pl.ds(toff, trp)], tbl.at[pl.ds(toff, trp)])
    rem = n - NS * trp
    if rem:
        @pl.when(s == NS - 1)
        def _():
            pltpu.sync_copy(
                g.at[pl.ds(NS * trp, rem)], tbl.at[pl.ds(NS * trp, rem)]
            )

    def fill(i, _):
        rows_a[i // (d // LANES), pl.ds((i % (d // LANES)) * LANES, LANES)] = (
            jnp.zeros((LANES,), jnp.float32)
        )
        return 0

    lax.fori_loop(0, K * d // LANES, fill, 0)
    zrows = (na // NS)
    for z in range(zrows // K):
        pltpu.sync_copy(
            rows_a, acc.at[pl.ds(pl.multiple_of(s * zrows + z * K, 8), K)]
        )
    if zrows % K:
        pltpu.sync_copy(
            rows_a.at[pl.ds(0, zrows % K)],
            acc.at[
                pl.ds(
                    pl.multiple_of(s * zrows + (zrows // K) * K, 8), zrows % K
                )
            ],
        )
    plsc.subcore_barrier()

    def eload(k, ebuf, esem):
        off = pl.multiple_of((s * nch + k) * 2 * K, 8)
        pltpu.async_copy(eidx.at[pl.ds(off, 2 * K)], ebuf, esem)

    def ewait(ebuf, esem):
        pltpu.make_async_copy(eidx.at[pl.ds(0, 2 * K)], ebuf, esem).wait()

    def remap(ebuf, didx):
        for v in range(K // LANES):
            x = ebuf[pl.ds(K + v * LANES, LANES)]
            local = x - base
            ok = (local >= 0) & (local < hn)
            didx[pl.ds(v * LANES, LANES)] = jnp.where(ok, local, hn)

    def gather(ebuf, rows, gsem):
        pltpu.async_copy(tbl.at[ebuf.at[pl.ds(0, K)]], rows, gsem)

    def gwait(ebuf, rows, gsem):
        pltpu.make_async_copy(tbl.at[ebuf.at[pl.ds(0, K)]], rows, gsem).wait()

    def scatter(rows, didx):
        pltpu.sync_copy(rows, acc.at[didx], add=True)

    eload(0, ebuf_a, esem_a)
    eload(1, ebuf_b, esem_b)

    def body(p, _):
        k0 = 2 * p
        ewait(ebuf_a, esem_a)
        remap(ebuf_a, didx_a)
        gather(ebuf_a, rows_a, gsem_a)
        ewait(ebuf_b, esem_b)
        remap(ebuf_b, didx_b)
        gather(ebuf_b, rows_b, gsem_b)
        gwait(ebuf_a, rows_a, gsem_a)
        scatter(rows_a, didx_a)

        @pl.when(p < nch // 2 - 1)
        def _():
            eload(k0 + 2, ebuf_a, esem_a)

        gwait(ebuf_b, rows_b, gsem_b)
        scatter(rows_b, didx_b)

        @pl.when(p < nch // 2 - 1)
        def _():
            eload(k0 + 3, ebuf_b, esem_b)

        return 0

    lax.fori_loop(0, nch // 2, body, 0)
    plsc.subcore_barrier()
    aoff = pl.multiple_of(s * arp, 8)
    pltpu.sync_copy(acc.at[pl.ds(aoff, arp)], out.at[c, pl.ds(aoff, arp)])


def _tca_body(x_ref, w_ref, degp_ref, g_ref):
    deg = degp_ref[0, :] + degp_ref[1, :] + 1.0
    dinv = lax.rsqrt(deg)[:, None]
    h = jnp.dot(x_ref[...], w_ref[...], preferred_element_type=jnp.float32)
    g_ref[...] = h * dinv


def _tcb_body(p_ref, g_ref, degp_ref, w_ref, b_ref, out_ref):
    deg = degp_ref[0, :] + degp_ref[1, :] + 1.0
    dinv = lax.rsqrt(deg)[:, None]
    t = dinv * (p_ref[...] + g_ref[...]) + b_ref[...]
    o1 = jnp.where(t >= 0, t, 0.01 * t)
    h2 = jnp.dot(o1, w_ref[...], preferred_element_type=jnp.float32)
    out_ref[...] = h2 * dinv


def _tcc_body(p_ref, g_ref, degp_ref, b_ref, out_ref):
    deg = degp_ref[0, :] + degp_ref[1, :] + 1.0
    dinv = lax.rsqrt(deg)[:, None]
    t = dinv * (p_ref[...] + g_ref[...]) + b_ref[...]
    out_ref[...] = jnp.where(t >= 0, t, 0.01 * t)


def kernel(x, edge_index, batch, W1, b1, W2, b2):
    n, d = x.shape
    e = edge_index.shape[1]

    n_acc = -(-n // (NS * KD)) * NS * KD
    hn = n_acc // 2
    na = hn + KD

    ept = -(-e // (NS * 2 * K)) * 2 * K
    e_pad = ept * NS
    src = edge_index[0].astype(jnp.int32)
    dst = edge_index[1].astype(jnp.int32)
    pad = e_pad - e
    srcp = jnp.concatenate([src, jnp.zeros((pad,), jnp.int32)])
    dstp = jnp.concatenate([dst, jnp.full((pad,), n, jnp.int32)])
    eidx = jnp.stack(
        [srcp.reshape(NS, ept // K, K), dstp.reshape(NS, ept // K, K)], axis=2
    ).reshape(-1)

    epd = -(-e // (NW * KD)) * KD
    e_pad_d = epd * NW
    pad_d = e_pad_d - e
    dst3 = jnp.concatenate([dst, jnp.full((pad_d,), n, jnp.int32)]).reshape(
        NW, epd // KD, KD
    )

    xp = jnp.concatenate([x, jnp.zeros((n_acc - n, d), x.dtype)])

    deg_kernel = pl.kernel(
        functools.partial(_deg_body, n_acc, epd),
        out_type=jax.ShapeDtypeStruct((NC, n_acc), jnp.float32),
        mesh=_mesh(),
        scratch_types={
            "didx": pltpu.VMEM((epd // KD, KD), jnp.int32),
            "ones_v": pltpu.VMEM((KD,), jnp.float32),
            "zvec": pltpu.VMEM((n_acc // NS,), jnp.float32),
            "deg_acc": pltpu.MemorySpace.VMEM_SHARED((n_acc,), jnp.float32),
        },
        name="gcn_sc_degree",
    )

    agg_kernel = pl.kernel(
        functools.partial(_agg_body, n, hn, na, ept, d),
        out_type=jax.ShapeDtypeStruct((NC, hn, d), jnp.float32),
        mesh=_mesh(),
        scratch_types={
            "ebuf_a": pltpu.VMEM((2 * K,), jnp.int32),
            "ebuf_b": pltpu.VMEM((2 * K,), jnp.int32),
            "didx_a": pltpu.VMEM((K,), jnp.int32),
            "didx_b": pltpu.VMEM((K,), jnp.int32),
            "rows_a": pltpu.VMEM((K, d), jnp.float32),
            "rows_b": pltpu.VMEM((K, d), jnp.float32),
            "esem_a": pltpu.SemaphoreType.DMA,
            "esem_b": pltpu.SemaphoreType.DMA,
            "gsem_a": pltpu.SemaphoreType.DMA,
            "gsem_b": pltpu.SemaphoreType.DMA,
            "tbl": pltpu.MemorySpace.VMEM_SHARED((n, d), jnp.float32),
            "acc": pltpu.MemorySpace.VMEM_SHARED((na, d), jnp.float32),
        },
        name="gcn_sc_scatter",
    )

    br = 2048
    grid = (n_acc // br,)
    tca = pl.pallas_call(
        _tca_body,
        grid=grid,
        in_specs=[
            pl.BlockSpec((br, d), lambda i: (i, 0)),
            pl.BlockSpec((d, d), lambda i: (0, 0)),
            pl.BlockSpec((NC, br), lambda i: (0, i)),
        ],
        out_specs=pl.BlockSpec((br, d), lambda i: (i, 0)),
        out_shape=jax.ShapeDtypeStruct((n_acc, d), jnp.float32),
        name="gcn_tc_g1",
    )
    tcb = pl.pallas_call(
        _tcb_body,
        grid=grid,
        in_specs=[
            pl.BlockSpec((br, d), lambda i: (i, 0)),
            pl.BlockSpec((br, d), lambda i: (i, 0)),
            pl.BlockSpec((NC, br), lambda i: (0, i)),
            pl.BlockSpec((d, d), lambda i: (0, 0)),
            pl.BlockSpec((1, d), lambda i: (0, 0)),
        ],
        out_specs=pl.BlockSpec((br, d), lambda i: (i, 0)),
        out_shape=jax.ShapeDtypeStruct((n_acc, d), jnp.float32),
        name="gcn_tc_layer1",
    )
    tcc = pl.pallas_call(
        _tcc_body,
        grid=grid,
        in_specs=[
            pl.BlockSpec((br, d), lambda i: (i, 0)),
            pl.BlockSpec((br, d), lambda i: (i, 0)),
            pl.BlockSpec((NC, br), lambda i: (0, i)),
            pl.BlockSpec((1, d), lambda i: (0, 0)),
        ],
        out_specs=pl.BlockSpec((br, d), lambda i: (i, 0)),
        out_shape=jax.ShapeDtypeStruct((n_acc, d), jnp.float32),
        name="gcn_tc_layer2",
    )

    degp = deg_kernel(dst3)
    g1 = tca(xp, W1, degp)
    p1 = agg_kernel(g1, eidx).reshape(n_acc, d)
    g2 = tcb(p1, g1, degp, W2, b1.reshape(1, d))
    p2 = agg_kernel(g2, eidx).reshape(n_acc, d)
    out = tcc(p2, g2, degp, b2.reshape(1, d))
    return out[:n]


# speedup vs baseline: 1.3888x; 1.1451x over previous
import functools

import jax
import jax.numpy as jnp
from jax import lax
from jax.experimental import pallas as pl
from jax.experimental.pallas import tpu as pltpu
from jax.experimental.pallas import tpu_sc as plsc

NC = 2
NS = 16
NW = NC * NS
LANES = 16
K = 32
KD = 128


def _mesh():
    return plsc.VectorSubcoreMesh(
        core_axis_name="c", subcore_axis_name="s", num_cores=NC, num_subcores=NS
    )


def _deg_body(n_acc, ep, dst3, out, didx, ones_v, zvec, deg_acc):
    c = lax.axis_index("c")
    s = lax.axis_index("s")
    wid = c * NS + s
    rp = n_acc // NS

    pltpu.sync_copy(dst3.at[wid], didx)

    def fill(i, _):
        zvec[pl.ds(i * LANES, LANES)] = jnp.zeros((LANES,), jnp.float32)
        ones_v[pl.ds((i % (KD // LANES)) * LANES, LANES)] = jnp.ones(
            (LANES,), jnp.float32
        )
        return 0

    lax.fori_loop(0, rp // LANES, fill, 0)
    pltpu.sync_copy(zvec, deg_acc.at[pl.ds(s * rp, rp)])
    plsc.subcore_barrier()

    def chunk(k, _):
        pltpu.sync_copy(ones_v, deg_acc.at[didx.at[k]], add=True)
        return 0

    lax.fori_loop(0, ep // KD, chunk, 0)
    plsc.subcore_barrier()
    pltpu.sync_copy(deg_acc.at[pl.ds(s * rp, rp)], out.at[c, pl.ds(s * rp, rp)])


def _agg_body(
    n, hn, na, ept, d, g, eidx, out,
    ebuf_a, ebuf_b, didx_a, didx_b, rows_a, rows_b,
    esem_a, esem_b, gsem_a, gsem_b, ssem_a, ssem_b, tbl, acc,
):
    c = lax.axis_index("c")
    s = lax.axis_index("s")
    nch = ept // K
    trp = (n // NS) // 8 * 8
    arp = hn // NS
    base = c * hn

    toff = pl.multiple_of(s * trp, 8)
    pltpu.sync_copy(g.at[pl.ds(toff, trp)], tbl.at[pl.ds(toff, trp)])
    rem = n - NS * trp
    if rem:
        @pl.when(s == NS - 1)
        def _():
            pltpu.sync_copy(
                g.at[pl.ds(NS * trp, rem)], tbl.at[pl.ds(NS * trp, rem)]
            )

    def fill(i, _):
        rows_a[i // (d // LANES), pl.ds((i % (d // LANES)) * LANES, LANES)] = (
            jnp.zeros((LANES,), jnp.float32)
        )
        return 0

    lax.fori_loop(0, K * d // LANES, fill, 0)
    zrows = (na // NS)
    for z in range(zrows // K):
        pltpu.sync_copy(
            rows_a, acc.at[pl.ds(pl.multiple_of(s * zrows + z * K, 8), K)]
        )
    if zrows % K:
        pltpu.sync_copy(
            rows_a.at[pl.ds(0, zrows % K)],
            acc.at[
                pl.ds(
                    pl.multiple_of(s * zrows + (zrows // K) * K, 8), zrows % K
                )
            ],
        )
    plsc.subcore_barrier()

    def eload(k, ebuf, esem):
        off = pl.multiple_of((s * nch + k) * 2 * K, 8)
        pltpu.async_copy(eidx.at[pl.ds(off, 2 * K)], ebuf, esem)

    def ewait(ebuf, esem):
        pltpu.make_async_copy(eidx.at[pl.ds(0, 2 * K)], ebuf, esem).wait()

    def remap(ebuf, didx):
        for v in range(K // LANES):
            x = ebuf[pl.ds(K + v * LANES, LANES)]
            local = x - base
            ok = (local >= 0) & (local < hn)
            didx[pl.ds(v * LANES, LANES)] = jnp.where(ok, local, hn)

    def gather(ebuf, rows, gsem):
        pltpu.async_copy(tbl.at[ebuf.at[pl.ds(0, K)]], rows, gsem)

    def gwait(ebuf, rows, gsem):
        pltpu.make_async_copy(tbl.at[ebuf.at[pl.ds(0, K)]], rows, gsem).wait()

    def scatter(rows, didx, ssem):
        pltpu.async_copy(rows, acc.at[didx], ssem, add=True)

    def swait(rows, didx, ssem):
        pltpu.make_async_copy(rows, acc.at[didx], ssem).wait()

    eload(0, ebuf_a, esem_a)
    eload(1, ebuf_b, esem_b)

    def side(p, k, ebuf, didx, rows, esem, gsem, ssem):
        ewait(ebuf, esem)

        @pl.when(p > 0)
        def _():
            swait(rows, didx, ssem)

        remap(ebuf, didx)
        gather(ebuf, rows, gsem)

    def tail(p, k, ebuf, didx, rows, esem, gsem, ssem):
        gwait(ebuf, rows, gsem)
        scatter(rows, didx, ssem)

        @pl.when(p < nch // 2 - 1)
        def _():
            eload(k + 2, ebuf, esem)

    def body(p, _):
        k0 = 2 * p
        side(p, k0, ebuf_a, didx_a, rows_a, esem_a, gsem_a, ssem_a)
        side(p, k0 + 1, ebuf_b, didx_b, rows_b, esem_b, gsem_b, ssem_b)
        tail(p, k0, ebuf_a, didx_a, rows_a, esem_a, gsem_a, ssem_a)
        tail(p, k0 + 1, ebuf_b, didx_b, rows_b, esem_b, gsem_b, ssem_b)
        return 0

    lax.fori_loop(0, nch // 2, body, 0)
    swait(rows_a, didx_a, ssem_a)
    swait(rows_b, didx_b, ssem_b)
    plsc.subcore_barrier()
    aoff = pl.multiple_of(s * arp, 8)
    pltpu.sync_copy(acc.at[pl.ds(aoff, arp)], out.at[c, pl.ds(aoff, arp)])


def _tca_body(x_ref, w_ref, degp_ref, g_ref):
    deg = degp_ref[0, :] + degp_ref[1, :] + 1.0
    dinv = lax.rsqrt(deg)[:, None]
    h = jnp.dot(x_ref[...], w_ref[...], preferred_element_type=jnp.float32)
    g_ref[...] = h * dinv


def _tcb_body(p_ref, g_ref, degp_ref, w_ref, b_ref, out_ref):
    deg = degp_ref[0, :] + degp_ref[1, :] + 1.0
    dinv = lax.rsqrt(deg)[:, None]
    t = dinv * (p_ref[...] + g_ref[...]) + b_ref[...]
    o1 = jnp.where(t >= 0, t, 0.01 * t)
    h2 = jnp.dot(o1, w_ref[...], preferred_element_type=jnp.float32)
    out_ref[...] = h2 * dinv


def _tcc_body(p_ref, g_ref, degp_ref, b_ref, out_ref):
    deg = degp_ref[0, :] + degp_ref[1, :] + 1.0
    dinv = lax.rsqrt(deg)[:, None]
    t = dinv * (p_ref[...] + g_ref[...]) + b_ref[...]
    out_ref[...] = jnp.where(t >= 0, t, 0.01 * t)


def kernel(x, edge_index, batch, W1, b1, W2, b2):
    n, d = x.shape
    e = edge_index.shape[1]

    n_acc = -(-n // (NS * KD)) * NS * KD
    hn = n_acc // 2
    na = hn + KD

    ept = -(-e // (NS * 2 * K)) * 2 * K
    e_pad = ept * NS
    src = edge_index[0].astype(jnp.int32)
    dst = edge_index[1].astype(jnp.int32)
    pad = e_pad - e
    srcp = jnp.concatenate([src, jnp.zeros((pad,), jnp.int32)])
    dstp = jnp.concatenate([dst, jnp.full((pad,), n, jnp.int32)])
    eidx = jnp.stack(
        [srcp.reshape(NS, ept // K, K), dstp.reshape(NS, ept // K, K)], axis=2
    ).reshape(-1)

    epd = -(-e // (NW * KD)) * KD
    e_pad_d = epd * NW
    pad_d = e_pad_d - e
    dst3 = jnp.concatenate([dst, jnp.full((pad_d,), n, jnp.int32)]).reshape(
        NW, epd // KD, KD
    )

    xp = jnp.concatenate([x, jnp.zeros((n_acc - n, d), x.dtype)])

    deg_kernel = pl.kernel(
        functools.partial(_deg_body, n_acc, epd),
        out_type=jax.ShapeDtypeStruct((NC, n_acc), jnp.float32),
        mesh=_mesh(),
        scratch_types={
            "didx": pltpu.VMEM((epd // KD, KD), jnp.int32),
            "ones_v": pltpu.VMEM((KD,), jnp.float32),
            "zvec": pltpu.VMEM((n_acc // NS,), jnp.float32),
            "deg_acc": pltpu.MemorySpace.VMEM_SHARED((n_acc,), jnp.float32),
        },
        name="gcn_sc_degree",
    )

    agg_kernel = pl.kernel(
        functools.partial(_agg_body, n, hn, na, ept, d),
        out_type=jax.ShapeDtypeStruct((NC, hn, d), jnp.float32),
        mesh=_mesh(),
        scratch_types={
            "ebuf_a": pltpu.VMEM((2 * K,), jnp.int32),
            "ebuf_b": pltpu.VMEM((2 * K,), jnp.int32),
            "didx_a": pltpu.VMEM((K,), jnp.int32),
            "didx_b": pltpu.VMEM((K,), jnp.int32),
            "rows_a": pltpu.VMEM((K, d), jnp.float32),
            "rows_b": pltpu.VMEM((K, d), jnp.float32),
            "esem_a": pltpu.SemaphoreType.DMA,
            "esem_b": pltpu.SemaphoreType.DMA,
            "gsem_a": pltpu.SemaphoreType.DMA,
            "gsem_b": pltpu.SemaphoreType.DMA,
            "ssem_a": pltpu.SemaphoreType.DMA,
            "ssem_b": pltpu.SemaphoreType.DMA,
            "tbl": pltpu.MemorySpace.VMEM_SHARED((n, d), jnp.float32),
            "acc": pltpu.MemorySpace.VMEM_SHARED((na, d), jnp.float32),
        },
        name="gcn_sc_scatter",
    )

    br = 2048
    grid = (n_acc // br,)
    tca = pl.pallas_call(
        _tca_body,
        grid=grid,
        in_specs=[
            pl.BlockSpec((br, d), lambda i: (i, 0)),
            pl.BlockSpec((d, d), lambda i: (0, 0)),
            pl.BlockSpec((NC, br), lambda i: (0, i)),
        ],
        out_specs=pl.BlockSpec((br, d), lambda i: (i, 0)),
        out_shape=jax.ShapeDtypeStruct((n_acc, d), jnp.float32),
        name="gcn_tc_g1",
    )
    tcb = pl.pallas_call(
        _tcb_body,
        grid=grid,
        in_specs=[
            pl.BlockSpec((br, d), lambda i: (i, 0)),
            pl.BlockSpec((br, d), lambda i: (i, 0)),
            pl.BlockSpec((NC, br), lambda i: (0, i)),
            pl.BlockSpec((d, d), lambda i: (0, 0)),
            pl.BlockSpec((1, d), lambda i: (0, 0)),
        ],
        out_specs=pl.BlockSpec((br, d), lambda i: (i, 0)),
        out_shape=jax.ShapeDtypeStruct((n_acc, d), jnp.float32),
        name="gcn_tc_layer1",
    )
    tcc = pl.pallas_call(
        _tcc_body,
        grid=grid,
        in_specs=[
            pl.BlockSpec((br, d), lambda i: (i, 0)),
            pl.BlockSpec((br, d), lambda i: (i, 0)),
            pl.BlockSpec((NC, br), lambda i: (0, i)),
            pl.BlockSpec((1, d), lambda i: (0, 0)),
        ],
        out_specs=pl.BlockSpec((br, d), lambda i: (i, 0)),
        out_shape=jax.ShapeDtypeStruct((n_acc, d), jnp.float32),
        name="gcn_tc_layer2",
    )

    degp = deg_kernel(dst3)
    g1 = tca(xp, W1, degp)
    p1 = agg_kernel(g1, eidx).reshape(n_acc, d)
    g2 = tcb(p1, g1, degp, W2, b1.reshape(1, d))
    p2 = agg_kernel(g2, eidx).reshape(n_acc, d)
    out = tcc(p2, g2, degp, b2.reshape(1, d))
    return out[:n]
